# baseline (speedup 1.0000x reference)
import functools

import jax
import jax.numpy as jnp
from jax import lax
from jax.experimental import pallas as pl
from jax.experimental.pallas import tpu as pltpu
from jax.experimental.pallas import tpu_sc as plsc

_NUM_EXPERTS = 8
_TOP_K = 2
_NC = 2
_NS = 16
_NL = 16


def _matmul_body(h_ref, w_ref, lg_ref):
    lg_ref[...] = lax.dot_general(
        w_ref[...], h_ref[...], (((1,), (1,)), ((), ())),
        preferred_element_type=jnp.float32)


def _router_logits(h, w, blk, off, n_chunk):
    off_blk = off // blk
    return pl.pallas_call(
        _matmul_body,
        grid=(n_chunk // blk,),
        in_specs=[
            pl.BlockSpec((blk, h.shape[1]), lambda i: (i + off_blk, 0)),
            pl.BlockSpec((_NUM_EXPERTS, h.shape[1]), lambda i: (0, 0)),
        ],
        out_specs=pl.BlockSpec((_NUM_EXPERTS, blk), lambda i: (0, i)),
        out_shape=jax.ShapeDtypeStruct((_NUM_EXPERTS, n_chunk), jnp.float32),
    )(h, w)


def _sc_route_body(lg_hbm, w1_hbm, se_hbm, pt_hbm,
                   lv, wv, sv, uv, sem_in, sem_w, sem_s, sem_u,
                   *, tpw, n_tokens):
    w = lax.axis_index("s") * _NC + lax.axis_index("c")
    base = w * tpw
    pltpu.async_copy(lg_hbm.at[:, pl.ds(base, tpw)], lv, sem_in).wait()

    zf = jnp.zeros((_NL,), jnp.float32)
    zi = jnp.zeros((_NL,), jnp.int32)

    def grp(g, accs):
        off = g * _NL
        ls = [lv[e, pl.ds(off, _NL)] for e in range(_NUM_EXPERTS)]

        m1, i1 = ls[0], zi
        m2, i2 = jnp.full((_NL,), -jnp.inf, jnp.float32), zi
        for e in range(1, _NUM_EXPERTS):
            v = ls[e]
            ei = jnp.full((_NL,), e, jnp.int32)
            gt1 = v > m1
            gt2 = v > m2
            m2 = jnp.where(gt1, m1, jnp.where(gt2, v, m2))
            i2 = jnp.where(gt1, i1, jnp.where(gt2, ei, i2))
            m1 = jnp.where(gt1, v, m1)
            i1 = jnp.where(gt1, ei, i1)

        w1 = 1.0 / (1.0 + jnp.exp(m2 - m1))

        exs = [jnp.exp(ls[e] - m1) for e in range(_NUM_EXPERTS)]
        s = exs[0]
        for e in range(1, _NUM_EXPERTS):
            s = s + exs[e]
        rinv = 1.0 / s
        accs = tuple(accs[e] + exs[e] * rinv for e in range(_NUM_EXPERTS))

        pair = i1 | (i2 << 16)

        wv[pl.ds(off, _NL)] = w1
        sv[pl.ds(off, _NL)] = pair
        return accs

    accs = lax.fori_loop(0, tpw // _NL, grp, (zf,) * _NUM_EXPERTS)
    for e in range(_NUM_EXPERTS):
        uv[pl.ds(e * _NL, _NL)] = accs[e]

    cw = pltpu.async_copy(wv, w1_hbm.at[pl.ds(base, tpw)], sem_w)
    cs = pltpu.async_copy(sv, se_hbm.at[pl.ds(base, tpw)], sem_s)
    cu = pltpu.async_copy(uv, pt_hbm.at[pl.ds(w * (_NUM_EXPERTS * _NL),
                                              _NUM_EXPERTS * _NL)], sem_u)
    cw.wait()
    cs.wait()
    cu.wait()


def _sc_route(lg_t, n_tokens):
    nw = _NC * _NS
    tpw = n_tokens // nw
    mesh = plsc.VectorSubcoreMesh(core_axis_name="c", subcore_axis_name="s",
                                  num_cores=_NC, num_subcores=_NS)
    return pl.kernel(
        functools.partial(_sc_route_body, tpw=tpw, n_tokens=n_tokens),
        out_type=(
            jax.ShapeDtypeStruct((n_tokens,), jnp.float32),
            jax.ShapeDtypeStruct((n_tokens,), jnp.int32),
            jax.ShapeDtypeStruct((nw * _NUM_EXPERTS * _NL,), jnp.float32),
        ),
        mesh=mesh,
        scratch_types=[
            pltpu.VMEM((_NUM_EXPERTS, tpw), jnp.float32),
            pltpu.VMEM((tpw,), jnp.float32),
            pltpu.VMEM((tpw,), jnp.int32),
            pltpu.VMEM((_NUM_EXPERTS * _NL,), jnp.float32),
            pltpu.SemaphoreType.DMA,
            pltpu.SemaphoreType.DMA,
            pltpu.SemaphoreType.DMA,
            pltpu.SemaphoreType.DMA,
        ],
    )(lg_t)


def _aux_body(pt0_ref, aux_ref, *, n_tokens):
    x = pt0_ref[...]
    col = jnp.sum(x, axis=0, keepdims=True)
    grp = lax.broadcasted_iota(jnp.int32, col.shape, 1) // _NL
    aux = 0.0
    for e in range(_NUM_EXPERTS):
        u_e = jnp.sum(jnp.where(grp == e, col, 0.0)) / n_tokens
        aux = aux + u_e * u_e
    aux_ref[0, 0] = _NUM_EXPERTS * aux


def _aux_loss(pt0, n_tokens):
    nw = _NC * _NS
    spec = pl.BlockSpec((nw, _NUM_EXPERTS * _NL), lambda: (0, 0))
    return pl.pallas_call(
        functools.partial(_aux_body, n_tokens=n_tokens),
        in_specs=[spec],
        out_specs=pl.BlockSpec(memory_space=pltpu.SMEM),
        out_shape=jax.ShapeDtypeStruct((1, 1), jnp.float32),
    )(pt0.reshape(nw, _NUM_EXPERTS * _NL))


@jax.jit
def kernel(hidden_states, gate_weight):
    b, t, hd = hidden_states.shape
    n_tokens = b * t
    h = hidden_states.reshape(n_tokens, hd)

    lg = _router_logits(h, gate_weight, 8192, 0, n_tokens)
    w1, pairs, pt = _sc_route(lg, n_tokens)
    aux = _aux_loss(pt, n_tokens)

    rw = jnp.abs(w1[:, None] - jnp.array([0.0, 1.0], jnp.float32))
    se = (pairs[:, None] >> jnp.array([0, 16], jnp.int32)) & 0xFFFF

    return rw, se, aux.reshape(())


# speedup vs baseline: 1.0535x; 1.0535x over previous
import functools

import jax
import jax.numpy as jnp
from jax import lax
from jax.experimental import pallas as pl
from jax.experimental.pallas import tpu as pltpu
from jax.experimental.pallas import tpu_sc as plsc

_NUM_EXPERTS = 8
_TOP_K = 2
_NC = 2
_NS = 16
_NL = 16


def _matmul_body(h_ref, w_ref, lg_ref):
    lg_ref[...] = lax.dot_general(
        w_ref[...], h_ref[...], (((1,), (1,)), ((), ())),
        preferred_element_type=jnp.float32)


def _router_logits(h, w, blk, off, n_chunk):
    off_blk = off // blk
    return pl.pallas_call(
        _matmul_body,
        grid=(n_chunk // blk,),
        in_specs=[
            pl.BlockSpec((blk, h.shape[1]), lambda i: (i + off_blk, 0)),
            pl.BlockSpec((_NUM_EXPERTS, h.shape[1]), lambda i: (0, 0)),
        ],
        out_specs=pl.BlockSpec((_NUM_EXPERTS, blk), lambda i: (0, i)),
        out_shape=jax.ShapeDtypeStruct((_NUM_EXPERTS, n_chunk), jnp.float32),
    )(h, w)


def _sc_route_body(lg_hbm, w1_hbm, se_hbm, pt_hbm,
                   lv, wv, sv, uv, sem_in, sem_w, sem_s, sem_u,
                   *, tpw, n_tokens):
    w = lax.axis_index("s") * _NC + lax.axis_index("c")
    base = w * tpw
    pltpu.async_copy(lg_hbm.at[:, pl.ds(base, tpw)], lv, sem_in).wait()

    zf = jnp.zeros((_NL,), jnp.float32)
    zi = jnp.zeros((_NL,), jnp.int32)

    def grp(g, accs):
        off = g * _NL
        ls = [lv[e, pl.ds(off, _NL)] for e in range(_NUM_EXPERTS)]

        m1, i1 = ls[0], zi
        m2, i2 = jnp.full((_NL,), -jnp.inf, jnp.float32), zi
        for e in range(1, _NUM_EXPERTS):
            v = ls[e]
            ei = jnp.full((_NL,), e, jnp.int32)
            gt1 = v > m1
            gt2 = v > m2
            m2 = jnp.where(gt1, m1, jnp.where(gt2, v, m2))
            i2 = jnp.where(gt1, i1, jnp.where(gt2, ei, i2))
            m1 = jnp.where(gt1, v, m1)
            i1 = jnp.where(gt1, ei, i1)

        w1 = 1.0 / (1.0 + jnp.exp(m2 - m1))

        exs = [jnp.exp(ls[e] - m1) for e in range(_NUM_EXPERTS)]
        s = exs[0]
        for e in range(1, _NUM_EXPERTS):
            s = s + exs[e]
        rinv = 1.0 / s
        accs = tuple(accs[e] + exs[e] * rinv for e in range(_NUM_EXPERTS))

        pair = i1 | (i2 << 16)

        wv[pl.ds(off, _NL)] = w1
        sv[pl.ds(off, _NL)] = pair
        return accs

    accs = lax.fori_loop(0, tpw // _NL, grp, (zf,) * _NUM_EXPERTS)
    for e in range(_NUM_EXPERTS):
        uv[pl.ds(e * _NL, _NL)] = accs[e]

    cw = pltpu.async_copy(wv, w1_hbm.at[pl.ds(base, tpw)], sem_w)
    cs = pltpu.async_copy(sv, se_hbm.at[pl.ds(base, tpw)], sem_s)
    cu = pltpu.async_copy(uv, pt_hbm.at[pl.ds(w * (_NUM_EXPERTS * _NL),
                                              _NUM_EXPERTS * _NL)], sem_u)
    cw.wait()
    cs.wait()
    cu.wait()


def _sc_route(lg_t, n_tokens):
    nw = _NC * _NS
    tpw = n_tokens // nw
    mesh = plsc.VectorSubcoreMesh(core_axis_name="c", subcore_axis_name="s",
                                  num_cores=_NC, num_subcores=_NS)
    return pl.kernel(
        functools.partial(_sc_route_body, tpw=tpw, n_tokens=n_tokens),
        out_type=(
            jax.ShapeDtypeStruct((n_tokens,), jnp.float32),
            jax.ShapeDtypeStruct((n_tokens,), jnp.int32),
            jax.ShapeDtypeStruct((nw * _NUM_EXPERTS * _NL,), jnp.float32),
        ),
        mesh=mesh,
        scratch_types=[
            pltpu.VMEM((_NUM_EXPERTS, tpw), jnp.float32),
            pltpu.VMEM((tpw,), jnp.float32),
            pltpu.VMEM((tpw,), jnp.int32),
            pltpu.VMEM((_NUM_EXPERTS * _NL,), jnp.float32),
            pltpu.SemaphoreType.DMA,
            pltpu.SemaphoreType.DMA,
            pltpu.SemaphoreType.DMA,
            pltpu.SemaphoreType.DMA,
        ],
    )(lg_t)


def _aux_body(pt0_ref, aux_ref, *, n_tokens):
    x = pt0_ref[...]
    col = jnp.sum(x, axis=0, keepdims=True)
    grp = lax.broadcasted_iota(jnp.int32, col.shape, 1) // _NL
    aux = 0.0
    for e in range(_NUM_EXPERTS):
        u_e = jnp.sum(jnp.where(grp == e, col, 0.0)) / n_tokens
        aux = aux + u_e * u_e
    aux_ref[0, 0] = _NUM_EXPERTS * aux


def _aux_loss(pt0, n_tokens):
    nw = _NC * _NS
    spec = pl.BlockSpec((nw, _NUM_EXPERTS * _NL), lambda: (0, 0))
    return pl.pallas_call(
        functools.partial(_aux_body, n_tokens=n_tokens),
        in_specs=[spec],
        out_specs=pl.BlockSpec(memory_space=pltpu.SMEM),
        out_shape=jax.ShapeDtypeStruct((1, 1), jnp.float32),
    )(pt0.reshape(nw, _NUM_EXPERTS * _NL))


@jax.jit
def kernel(hidden_states, gate_weight):
    b, t, hd = hidden_states.shape
    n_tokens = b * t
    h = hidden_states.reshape(n_tokens, hd)

    lg = _router_logits(h, gate_weight, 4096, 0, n_tokens)
    w1, pairs, pt = _sc_route(lg, n_tokens)
    aux = _aux_loss(pt, n_tokens)

    rw = jnp.abs(w1[:, None] - jnp.array([0.0, 1.0], jnp.float32))
    se = (pairs[:, None] >> jnp.array([0, 16], jnp.int32)) & 0xFFFF

    return rw, se, aux.reshape(())
